# Initial kernel scaffold; baseline (speedup 1.0000x reference)
import jax
import jax.numpy as jnp
from jax.experimental import pallas as pl


def kernel(feature, label):
    raise NotImplementedError("write your pallas kernel here")


# speedup vs baseline: 1.6659x; 1.6659x over previous
import functools

import jax
import jax.numpy as jnp
from jax import lax
from jax.experimental import pallas as pl
from jax.experimental.pallas import tpu as pltpu

LANES = 128
M = 32
P = 4
MARGIN = 0.2


def _quad_kernel(bq, f_ref, lrow_ref, lcol_ref, lst_ref, out_ref):
    rid = lax.broadcasted_iota(jnp.int32, (LANES, LANES), 0)
    cid = lax.broadcasted_iota(jnp.int32, (LANES, LANES), 1)
    eye = rid == cid
    sg = (rid // M) == (cid // M)
    sg_f = sg.astype(jnp.float32)
    lg = lax.broadcasted_iota(jnp.int32, (M, LANES), 1) // M

    def gsum_row(r):
        return lax.dot_general(r, sg_f, (((1,), (0,)), ((), ())),
                               preferred_element_type=jnp.float32)

    def gsum_col(c):
        return lax.dot_general(c, sg_f, (((0,), (0,)), ((), ())),
                               preferred_element_type=jnp.float32)

    for q in range(bq):
        X = f_ref[q]
        lrow = lrow_ref[q]
        lcol = lcol_ref[q]
        lst = lst_ref[q]

        gram = lax.dot_general(X, X, (((1,), (1,)), ((), ())),
                               preferred_element_type=jnp.float32)
        ns_row = jnp.sum(jnp.where(eye, gram, 0.0), axis=0, keepdims=True)
        ns_col = jnp.sum(X * X, axis=1, keepdims=True)
        dsq = jnp.maximum(ns_col + ns_row - 2.0 * gram, 0.0)

        hp_full = lrow == lcol
        hp_m = hp_full & sg
        hn_m = jnp.logical_not(hp_full) & sg
        maxsq = jnp.max(jnp.where(hp_m, dsq, -jnp.inf), axis=1, keepdims=True)
        minsq = jnp.min(jnp.where(hn_m, dsq, jnp.inf), axis=1, keepdims=True)
        hard_col = jnp.maximum(MARGIN + jnp.sqrt(maxsq) - jnp.sqrt(minsq), 0.0)

        dsq_st = (jnp.where(lg == 0, dsq[0 * M:1 * M, :], 0.0)
                  + jnp.where(lg == 1, dsq[1 * M:2 * M, :], 0.0)
                  + jnp.where(lg == 2, dsq[2 * M:3 * M, :], 0.0)
                  + jnp.where(lg == 3, dsq[3 * M:4 * M, :], 0.0))
        dist = jnp.sqrt(dsq_st)
        hpf = (lst == lrow).astype(jnp.float32)
        hnf = 1.0 - hpf

        margin_plus = MARGIN + dist
        s_mat = jnp.zeros((M, LANES), jnp.float32)
        c_mat = jnp.zeros((M, LANES), jnp.float32)
        for k in range(M):
            row_d = dist[k:k + 1, :]
            row_n = hnf[k:k + 1, :]
            v = hpf * jnp.maximum(margin_plus - row_d, 0.0) * row_n
            s_mat = s_mat + v
            c_mat = c_mat + (v > 0.0).astype(jnp.float32)
        full_sum_row = jnp.sum(s_mat, axis=0, keepdims=True)
        full_num_row = jnp.sum(c_mat, axis=0, keepdims=True)
        dist_sum_row = jnp.sum(dist, axis=0, keepdims=True)

        mx = jnp.max(X, axis=1, keepdims=True)
        lse = mx + jnp.log(jnp.sum(jnp.exp(X - mx), axis=1, keepdims=True))
        true_logit = jnp.sum(jnp.where(cid == lcol, X, 0.0), axis=1,
                             keepdims=True)
        ce_col = lse - true_logit
        pred = jnp.min(jnp.where(X == mx, cid, LANES), axis=1, keepdims=True)
        cor_col = (pred == lcol).astype(jnp.float32)

        packed = jnp.concatenate([
            gsum_row(full_sum_row),
            gsum_row(full_num_row),
            gsum_row(dist_sum_row),
            gsum_col(hard_col),
            gsum_col(ce_col),
            gsum_col(cor_col),
            jnp.zeros((2, LANES), jnp.float32),
        ], axis=0)
        out_ref[q] = packed


@jax.jit
def kernel(feature, label):
    n, m, d = feature.shape
    feature = feature.astype(jnp.float32)
    label = label.astype(jnp.int32)
    g = n // P

    f2 = feature.reshape(g, P * m, d)
    lrow = label.reshape(g, 1, P * m)
    lcol = label.reshape(g, P * m, 1)
    lst = jnp.repeat(label.reshape(g, P, m).transpose(0, 2, 1), m, axis=2)

    bq = 1
    for cand in (8, 4, 2):
        if g % cand == 0:
            bq = cand
            break

    out = pl.pallas_call(
        functools.partial(_quad_kernel, bq),
        out_shape=jax.ShapeDtypeStruct((g, 8, LANES), jnp.float32),
        grid=(g // bq,),
        in_specs=[
            pl.BlockSpec((bq, P * m, d), lambda i: (i, 0, 0)),
            pl.BlockSpec((bq, 1, P * m), lambda i: (i, 0, 0)),
            pl.BlockSpec((bq, P * m, 1), lambda i: (i, 0, 0)),
            pl.BlockSpec((bq, m, P * m), lambda i: (i, 0, 0)),
        ],
        out_specs=pl.BlockSpec((bq, 8, LANES), lambda i: (i, 0, 0)),
        compiler_params=pltpu.CompilerParams(
            dimension_semantics=("parallel",)),
    )(f2, lrow, lcol, lst)

    met = out[:, :, ::m]
    full_sum = met[:, 0, :].reshape(n)
    full_num = met[:, 1, :].reshape(n)
    dist_sum = met[:, 2, :].reshape(n)
    hard_sum = met[:, 3, :].reshape(n)
    ce_sum = met[:, 4, :]
    cor_sum = met[:, 5, :]

    full_mean = jnp.where(full_num > 0.0,
                          full_sum / jnp.maximum(full_num, 1.0), 0.0)
    hard_mean = hard_sum / m
    mean_dist = dist_sum / (m * m)
    entropy_loss = jnp.sum(ce_sum) / (n * m)
    accuracy = jnp.sum(cor_sum) / (n * m + 0.0001)
    return full_mean, hard_mean, mean_dist, full_num, entropy_loss, accuracy
